# Initial kernel scaffold; baseline (speedup 1.0000x reference)
import functools

import jax
import jax.numpy as jnp
from jax import lax
from jax.experimental import pallas as pl
from jax.experimental.pallas import tpu as pltpu
from jax.experimental.pallas import tpu_sc as plsc

N = 10000
E = 160000
D = 256
H = 128
NC = 2
NS = 16
EPT = E // NS
CH = 128
FULL_CHUNKS = EPT // CH
REM = EPT - FULL_CHUNKS * CH
RPT = N // NS
CW = 16


def _proj_body(x_ref, wp_ref, bp_ref, lo_ref, hi_ref):
    xp = jnp.maximum(
        jnp.dot(x_ref[...], wp_ref[...], preferred_element_type=jnp.float32)
        + bp_ref[...],
        0.0,
    )
    lo_ref[...] = xp[:, :H]
    hi_ref[...] = xp[:, H:]


def _combine_body(agg_ref, cnt_ref, x_ref, wl_ref, bl_ref, wr_ref, o_ref):
    inv = 1.0 / jnp.maximum(cnt_ref[:, 0:1], 1.0)
    m0 = agg_ref[0] * inv
    m1 = agg_ref[1] * inv
    o_ref[...] = (
        jnp.dot(m0, wl_ref[:H, :], preferred_element_type=jnp.float32)
        + jnp.dot(m1, wl_ref[H:, :], preferred_element_type=jnp.float32)
        + jnp.dot(x_ref[...], wr_ref[...], preferred_element_type=jnp.float32)
        + bl_ref[...]
    )


def _sc_body(xp_lo, xp_hi, src_hbm, dst_hbm, z_agg, z_cnt, ones_hbm,
             agg_out, cnt_out,
             src_v, dst_v, src_r, dst_r, rows_v, rows_r, ones_v, ones_r,
             agg_sh, cnt_sh, sem):
    cid = lax.axis_index("c")
    sid = lax.axis_index("s")

    r0 = sid * RPT
    pltpu.sync_copy(z_agg, agg_sh.at[pl.ds(r0, RPT)])

    @pl.when(cid == 0)
    def _():
        pltpu.sync_copy(z_cnt, cnt_sh.at[pl.ds(r0, RPT)])
        pltpu.sync_copy(ones_hbm, ones_v)
        pltpu.sync_copy(ones_hbm.at[pl.ds(0, REM)], ones_r)

    plsc.subcore_barrier()

    base = sid * EPT

    @pl.loop(0, FULL_CHUNKS)
    def _(j):
        off = base + j * CH
        pltpu.sync_copy(src_hbm.at[pl.ds(off, CH)], src_v)
        pltpu.sync_copy(dst_hbm.at[pl.ds(off, CH)], dst_v)

        @pl.when(cid == 0)
        def _():
            pltpu.async_copy(xp_lo.at[src_v], rows_v, sem).wait()
            pltpu.sync_copy(ones_v, cnt_sh.at[dst_v], add=True)

        @pl.when(cid == 1)
        def _():
            pltpu.async_copy(xp_hi.at[src_v], rows_v, sem).wait()

        pltpu.sync_copy(rows_v, agg_sh.at[dst_v], add=True)

    offr = base + FULL_CHUNKS * CH
    pltpu.sync_copy(src_hbm.at[pl.ds(offr, REM)], src_r)
    pltpu.sync_copy(dst_hbm.at[pl.ds(offr, REM)], dst_r)

    @pl.when(cid == 0)
    def _():
        pltpu.async_copy(xp_lo.at[src_r], rows_r, sem).wait()
        pltpu.sync_copy(ones_r, cnt_sh.at[dst_r], add=True)

    @pl.when(cid == 1)
    def _():
        pltpu.async_copy(xp_hi.at[src_r], rows_r, sem).wait()

    pltpu.sync_copy(rows_r, agg_sh.at[dst_r], add=True)

    plsc.subcore_barrier()

    @pl.when(cid == 0)
    def _():
        pltpu.sync_copy(agg_sh.at[pl.ds(r0, RPT)], agg_out.at[0, pl.ds(r0, RPT)])
        pltpu.sync_copy(cnt_sh.at[pl.ds(r0, RPT)], cnt_out.at[pl.ds(r0, RPT)])

    @pl.when(cid == 1)
    def _():
        pltpu.sync_copy(agg_sh.at[pl.ds(r0, RPT)], agg_out.at[1, pl.ds(r0, RPT)])


_sc_call = pl.kernel(
    _sc_body,
    out_type=[
        jax.ShapeDtypeStruct((NC, N, H), jnp.float32),
        jax.ShapeDtypeStruct((N, CW), jnp.float32),
    ],
    mesh=plsc.VectorSubcoreMesh(core_axis_name="c", subcore_axis_name="s"),
    scratch_types=[
        pltpu.VMEM((CH,), jnp.int32),
        pltpu.VMEM((CH,), jnp.int32),
        pltpu.VMEM((REM,), jnp.int32),
        pltpu.VMEM((REM,), jnp.int32),
        pltpu.VMEM((CH, H), jnp.float32),
        pltpu.VMEM((REM, H), jnp.float32),
        pltpu.VMEM((CH, CW), jnp.float32),
        pltpu.VMEM((REM, CW), jnp.float32),
        pltpu.VMEM_SHARED((N, H), jnp.float32),
        pltpu.VMEM_SHARED((N, CW), jnp.float32),
        pltpu.SemaphoreType.DMA,
    ],
)

_ROWS = 1000
_GRID = N // _ROWS

_proj_call = pl.pallas_call(
    _proj_body,
    grid=(_GRID,),
    in_specs=[
        pl.BlockSpec((_ROWS, D), lambda i: (i, 0)),
        pl.BlockSpec((D, D), lambda i: (0, 0)),
        pl.BlockSpec((1, D), lambda i: (0, 0)),
    ],
    out_specs=[
        pl.BlockSpec((_ROWS, H), lambda i: (i, 0)),
        pl.BlockSpec((_ROWS, H), lambda i: (i, 0)),
    ],
    out_shape=[
        jax.ShapeDtypeStruct((N, H), jnp.float32),
        jax.ShapeDtypeStruct((N, H), jnp.float32),
    ],
)

_combine_call = pl.pallas_call(
    _combine_body,
    grid=(_GRID,),
    in_specs=[
        pl.BlockSpec((NC, _ROWS, H), lambda i: (0, i, 0)),
        pl.BlockSpec((_ROWS, CW), lambda i: (i, 0)),
        pl.BlockSpec((_ROWS, D), lambda i: (i, 0)),
        pl.BlockSpec((D, D), lambda i: (0, 0)),
        pl.BlockSpec((1, D), lambda i: (0, 0)),
        pl.BlockSpec((D, D), lambda i: (0, 0)),
    ],
    out_specs=pl.BlockSpec((_ROWS, D), lambda i: (i, 0)),
    out_shape=jax.ShapeDtypeStruct((N, D), jnp.float32),
)


@jax.jit
def kernel(x, ei, W_proj, b_proj, W_l, b_l, W_r):
    xp_lo, xp_hi = _proj_call(x, W_proj, b_proj.reshape(1, D))
    z_agg = jnp.zeros((RPT, H), jnp.float32)
    z_cnt = jnp.zeros((RPT, CW), jnp.float32)
    ones = jnp.ones((CH, CW), jnp.float32)
    agg, cnt = _sc_call(xp_lo, xp_hi, ei[0], ei[1], z_agg, z_cnt, ones)
    return _combine_call(agg, cnt, x, W_l, b_l.reshape(1, D), W_r)


# speedup vs baseline: 3.8476x; 3.8476x over previous
import jax
import jax.numpy as jnp
from jax import lax
from jax.experimental import pallas as pl
from jax.experimental.pallas import tpu as pltpu
from jax.experimental.pallas import tpu_sc as plsc

N = 10000
E = 160000
D = 256
H = 128
NC = 2
NS = 16
L = 16
EPT = E // NS
CH = 128
FULL_CHUNKS = EPT // CH
REM = EPT - FULL_CHUNKS * CH
RPT = 624
TAIL = N - NS * RPT
TAIL0 = N - TAIL
RPW = 312
NW = NC * NS


def _proj_body(x_ref, wp_ref, bp_ref, o_ref):
    xp = jnp.maximum(
        jnp.dot(x_ref[...], wp_ref[...], preferred_element_type=jnp.float32)
        + bp_ref[...],
        0.0,
    )
    o_ref[0] = xp[:, :H]
    o_ref[1] = xp[:, H:]


def _combine_body(agg_lo_ref, agg_hi_ref, cnt_ref, x_ref, wl_ref, bl_ref,
                  wr_ref, o_ref):
    inv = 1.0 / jnp.maximum(cnt_ref[:, 0:1], 1.0)
    m0 = agg_lo_ref[...] * inv
    m1 = agg_hi_ref[...] * inv
    o_ref[...] = (
        jnp.dot(m0, wl_ref[:H, :], preferred_element_type=jnp.float32)
        + jnp.dot(m1, wl_ref[H:, :], preferred_element_type=jnp.float32)
        + jnp.dot(x_ref[...], wr_ref[...], preferred_element_type=jnp.float32)
        + bl_ref[...]
    )


def _sc_body(xp_hbm, src_hbm, dst_hbm, z_agg, ones_hbm,
             agg_out, cnt_out,
             src_v, dst_v, src_r, dst_r, rows_v, rows_r,
             agg_sh, sem):
    cid = lax.axis_index("c")
    sid = lax.axis_index("s")
    coff = jnp.full((L,), cid * N, jnp.int32)

    r0 = sid * RPT
    pltpu.sync_copy(z_agg, agg_sh.at[pl.ds(r0, RPT)])

    @pl.when(sid == 0)
    def _():
        pltpu.sync_copy(z_agg.at[pl.ds(0, TAIL)], agg_sh.at[pl.ds(TAIL0, TAIL)])

    plsc.subcore_barrier()

    base = sid * EPT

    @pl.loop(0, FULL_CHUNKS)
    def _(j):
        off = base + j * CH
        pltpu.sync_copy(src_hbm.at[pl.ds(off, CH)], src_v)
        pltpu.sync_copy(dst_hbm.at[pl.ds(off, CH)], dst_v)
        for k in range(CH // L):
            sl = pl.ds(k * L, L)
            src_v[sl] = src_v[sl] + coff
        pltpu.async_copy(xp_hbm.at[src_v], rows_v, sem).wait()
        pltpu.sync_copy(rows_v, agg_sh.at[dst_v], add=True)

    offr = base + FULL_CHUNKS * CH
    pltpu.sync_copy(src_hbm.at[pl.ds(offr, REM)], src_r)
    pltpu.sync_copy(dst_hbm.at[pl.ds(offr, REM)], dst_r)
    src_r[pl.ds(0, L)] = src_r[pl.ds(0, L)] + coff
    pltpu.async_copy(xp_hbm.at[src_r], rows_r, sem).wait()
    pltpu.sync_copy(rows_r, agg_sh.at[dst_r], add=True)

    plsc.subcore_barrier()

    o0 = cid * N + r0
    pltpu.sync_copy(agg_sh.at[pl.ds(r0, RPT)], agg_out.at[pl.ds(o0, RPT)])

    @pl.when(sid == 0)
    def _():
        pltpu.sync_copy(agg_sh.at[pl.ds(TAIL0, TAIL)],
                        agg_out.at[pl.ds(cid * N + TAIL0, TAIL)])

    plsc.subcore_barrier()

    pltpu.sync_copy(z_agg, agg_sh.at[pl.ds(r0, RPT)])

    @pl.when(sid == 0)
    def _():
        pltpu.sync_copy(z_agg.at[pl.ds(0, TAIL)], agg_sh.at[pl.ds(TAIL0, TAIL)])

    pltpu.sync_copy(ones_hbm, rows_v)
    pltpu.sync_copy(ones_hbm.at[pl.ds(0, REM)], rows_r)
    plsc.subcore_barrier()

    @pl.loop(0, FULL_CHUNKS)
    def _(j):
        off = base + j * CH
        pltpu.sync_copy(dst_hbm.at[pl.ds(off, CH)], dst_v)
        pltpu.sync_copy(rows_v, agg_sh.at[dst_v], add=True)

    pltpu.sync_copy(dst_hbm.at[pl.ds(offr, REM)], dst_r)
    pltpu.sync_copy(rows_r, agg_sh.at[dst_r], add=True)

    plsc.subcore_barrier()

    w = cid * NS + sid
    c0 = w * RPW
    pltpu.sync_copy(agg_sh.at[pl.ds(c0, RPW)], cnt_out.at[pl.ds(c0, RPW)])

    @pl.when(w == 0)
    def _():
        pltpu.sync_copy(agg_sh.at[pl.ds(TAIL0, TAIL)],
                        cnt_out.at[pl.ds(TAIL0, TAIL)])


_sc_call = pl.kernel(
    _sc_body,
    out_type=[
        jax.ShapeDtypeStruct((NC * N, H), jnp.float32),
        jax.ShapeDtypeStruct((N, H), jnp.float32),
    ],
    mesh=plsc.VectorSubcoreMesh(core_axis_name="c", subcore_axis_name="s"),
    scratch_types=[
        pltpu.VMEM((CH,), jnp.int32),
        pltpu.VMEM((CH,), jnp.int32),
        pltpu.VMEM((REM,), jnp.int32),
        pltpu.VMEM((REM,), jnp.int32),
        pltpu.VMEM((CH, H), jnp.float32),
        pltpu.VMEM((REM, H), jnp.float32),
        pltpu.VMEM_SHARED((N, H), jnp.float32),
        pltpu.SemaphoreType.DMA,
    ],
)

_ROWS = 1000
_GRID = N // _ROWS

_proj_call = pl.pallas_call(
    _proj_body,
    grid=(_GRID,),
    in_specs=[
        pl.BlockSpec((_ROWS, D), lambda i: (i, 0)),
        pl.BlockSpec((D, D), lambda i: (0, 0)),
        pl.BlockSpec((1, D), lambda i: (0, 0)),
    ],
    out_specs=pl.BlockSpec((NC, _ROWS, H), lambda i: (0, i, 0)),
    out_shape=jax.ShapeDtypeStruct((NC, N, H), jnp.float32),
)

_combine_call = pl.pallas_call(
    _combine_body,
    grid=(_GRID,),
    in_specs=[
        pl.BlockSpec((_ROWS, H), lambda i: (i, 0)),
        pl.BlockSpec((_ROWS, H), lambda i: (i + _GRID, 0)),
        pl.BlockSpec((_ROWS, H), lambda i: (i, 0)),
        pl.BlockSpec((_ROWS, D), lambda i: (i, 0)),
        pl.BlockSpec((D, D), lambda i: (0, 0)),
        pl.BlockSpec((1, D), lambda i: (0, 0)),
        pl.BlockSpec((D, D), lambda i: (0, 0)),
    ],
    out_specs=pl.BlockSpec((_ROWS, D), lambda i: (i, 0)),
    out_shape=jax.ShapeDtypeStruct((N, D), jnp.float32),
)


@jax.jit
def kernel(x, ei, W_proj, b_proj, W_l, b_l, W_r):
    xp = _proj_call(x, W_proj, b_proj.reshape(1, D))
    xp_flat = xp.reshape(NC * N, H)
    z_agg = jnp.zeros((RPT, H), jnp.float32)
    ones = jnp.ones((CH, H), jnp.float32)
    agg, cnt = _sc_call(xp_flat, ei[0], ei[1], z_agg, ones)
    return _combine_call(agg, agg, cnt, x, W_l, b_l.reshape(1, D), W_r)


# speedup vs baseline: 4.2581x; 1.1067x over previous
import jax
import jax.numpy as jnp
from jax import lax
from jax.experimental import pallas as pl
from jax.experimental.pallas import tpu as pltpu
from jax.experimental.pallas import tpu_sc as plsc

N = 10000
E = 160000
D = 256
H = 128
NC = 2
NS = 16
L = 16
CH = 128
NCH = E // CH
CPT1 = NCH // NS
XTRA1 = NCH - CPT1 * NS
CPC2 = NCH // NC
CPT2 = CPC2 // NS
XTRA2 = CPC2 - CPT2 * NS
RPT = 624
TAIL = N - NS * RPT
TAIL0 = N - TAIL


def _proj_body(x_ref, wp_ref, bp_ref, o_ref):
    xp = jnp.maximum(
        jnp.dot(x_ref[...], wp_ref[...], preferred_element_type=jnp.float32)
        + bp_ref[...],
        0.0,
    )
    o_ref[0] = xp[:, :H]
    o_ref[1] = xp[:, H:]


def _combine_body(agg_lo_ref, agg_hi_ref, cnt0_ref, cnt1_ref, x_ref, wl_ref,
                  bl_ref, wr_ref, o_ref):
    cnt = cnt0_ref[:, 0:1] + cnt1_ref[:, 0:1]
    inv = 1.0 / jnp.maximum(cnt, 1.0)
    m0 = agg_lo_ref[...] * inv
    m1 = agg_hi_ref[...] * inv
    o_ref[...] = (
        jnp.dot(m0, wl_ref[:H, :], preferred_element_type=jnp.float32)
        + jnp.dot(m1, wl_ref[H:, :], preferred_element_type=jnp.float32)
        + jnp.dot(x_ref[...], wr_ref[...], preferred_element_type=jnp.float32)
        + bl_ref[...]
    )


def _zero_agg(sid, z_agg, agg_sh):
    pltpu.sync_copy(z_agg, agg_sh.at[pl.ds(sid * RPT, RPT)])

    @pl.when(sid == 0)
    def _():
        pltpu.sync_copy(z_agg.at[pl.ds(0, TAIL)], agg_sh.at[pl.ds(TAIL0, TAIL)])


def _sc_body(xp_hbm, src_hbm, dst_hbm, z_agg, ones_hbm,
             agg_out, cnt_out,
             src_v, dst_v, rows_v, agg_sh, sem):
    cid = lax.axis_index("c")
    sid = lax.axis_index("s")
    coff = jnp.full((L,), cid * N, jnp.int32)

    def gather_scatter_chunk(chunk):
        off = chunk * CH
        pltpu.sync_copy(src_hbm.at[pl.ds(off, CH)], src_v)
        pltpu.sync_copy(dst_hbm.at[pl.ds(off, CH)], dst_v)
        for k in range(CH // L):
            sl = pl.ds(k * L, L)
            src_v[sl] = src_v[sl] + coff
        pltpu.async_copy(xp_hbm.at[src_v], rows_v, sem).wait()
        pltpu.sync_copy(rows_v, agg_sh.at[dst_v], add=True)

    def count_chunk(chunk):
        off = chunk * CH
        pltpu.sync_copy(dst_hbm.at[pl.ds(off, CH)], dst_v)
        pltpu.sync_copy(rows_v, agg_sh.at[dst_v], add=True)

    _zero_agg(sid, z_agg, agg_sh)
    plsc.subcore_barrier()

    @pl.loop(0, CPT1)
    def _(j):
        gather_scatter_chunk(sid * CPT1 + j)

    @pl.when(sid < XTRA1)
    def _():
        gather_scatter_chunk(NS * CPT1 + sid)

    plsc.subcore_barrier()

    r0 = sid * RPT
    o0 = cid * N + r0
    pltpu.sync_copy(agg_sh.at[pl.ds(r0, RPT)], agg_out.at[pl.ds(o0, RPT)])

    @pl.when(sid == 0)
    def _():
        pltpu.sync_copy(agg_sh.at[pl.ds(TAIL0, TAIL)],
                        agg_out.at[pl.ds(cid * N + TAIL0, TAIL)])

    plsc.subcore_barrier()

    _zero_agg(sid, z_agg, agg_sh)
    pltpu.sync_copy(ones_hbm, rows_v)
    plsc.subcore_barrier()

    c2 = cid * CPC2 + sid * CPT2

    @pl.loop(0, CPT2)
    def _(j):
        count_chunk(c2 + j)

    @pl.when(sid < XTRA2)
    def _():
        count_chunk(cid * CPC2 + NS * CPT2 + sid)

    plsc.subcore_barrier()

    pltpu.sync_copy(agg_sh.at[pl.ds(r0, RPT)], cnt_out.at[pl.ds(o0, RPT)])

    @pl.when(sid == 0)
    def _():
        pltpu.sync_copy(agg_sh.at[pl.ds(TAIL0, TAIL)],
                        cnt_out.at[pl.ds(cid * N + TAIL0, TAIL)])


_sc_call = pl.kernel(
    _sc_body,
    out_type=[
        jax.ShapeDtypeStruct((NC * N, H), jnp.float32),
        jax.ShapeDtypeStruct((NC * N, H), jnp.float32),
    ],
    mesh=plsc.VectorSubcoreMesh(core_axis_name="c", subcore_axis_name="s"),
    scratch_types=[
        pltpu.VMEM((CH,), jnp.int32),
        pltpu.VMEM((CH,), jnp.int32),
        pltpu.VMEM((CH, H), jnp.float32),
        pltpu.VMEM_SHARED((N, H), jnp.float32),
        pltpu.SemaphoreType.DMA,
    ],
)

_ROWS = 1000
_GRID = N // _ROWS

_proj_call = pl.pallas_call(
    _proj_body,
    grid=(_GRID,),
    in_specs=[
        pl.BlockSpec((_ROWS, D), lambda i: (i, 0)),
        pl.BlockSpec((D, D), lambda i: (0, 0)),
        pl.BlockSpec((1, D), lambda i: (0, 0)),
    ],
    out_specs=pl.BlockSpec((NC, _ROWS, H), lambda i: (0, i, 0)),
    out_shape=jax.ShapeDtypeStruct((NC, N, H), jnp.float32),
)

_combine_call = pl.pallas_call(
    _combine_body,
    grid=(_GRID,),
    in_specs=[
        pl.BlockSpec((_ROWS, H), lambda i: (i, 0)),
        pl.BlockSpec((_ROWS, H), lambda i: (i + _GRID, 0)),
        pl.BlockSpec((_ROWS, H), lambda i: (i, 0)),
        pl.BlockSpec((_ROWS, H), lambda i: (i + _GRID, 0)),
        pl.BlockSpec((_ROWS, D), lambda i: (i, 0)),
        pl.BlockSpec((D, D), lambda i: (0, 0)),
        pl.BlockSpec((1, D), lambda i: (0, 0)),
        pl.BlockSpec((D, D), lambda i: (0, 0)),
    ],
    out_specs=pl.BlockSpec((_ROWS, D), lambda i: (i, 0)),
    out_shape=jax.ShapeDtypeStruct((N, D), jnp.float32),
)


@jax.jit
def kernel(x, ei, W_proj, b_proj, W_l, b_l, W_r):
    xp = _proj_call(x, W_proj, b_proj.reshape(1, D))
    xp_flat = xp.reshape(NC * N, H)
    z_agg = jnp.zeros((RPT, H), jnp.float32)
    ones = jnp.ones((CH, H), jnp.float32)
    agg, cnt = _sc_call(xp_flat, ei[0], ei[1], z_agg, ones)
    return _combine_call(agg, agg, cnt, cnt, x, W_l, b_l.reshape(1, D), W_r)


# speedup vs baseline: 5.7622x; 1.3532x over previous
import jax
import jax.numpy as jnp
from jax import lax
from jax.experimental import pallas as pl
from jax.experimental.pallas import tpu as pltpu
from jax.experimental.pallas import tpu_sc as plsc

N = 10000
E = 160000
D = 256
H = 128
NC = 2
NS = 16
L = 16
CH = 128
NCH = E // CH
CPT1 = NCH // NS
XTRA1 = NCH - CPT1 * NS
CPC2 = NCH // NC
CPT2 = CPC2 // NS
XTRA2 = CPC2 - CPT2 * NS
RPT = 624
TAIL = N - NS * RPT
TAIL0 = N - TAIL


def _proj_body(x_ref, wp_ref, bp_ref, o_ref):
    xp = jnp.maximum(
        jnp.dot(x_ref[...], wp_ref[...], preferred_element_type=jnp.float32)
        + bp_ref[...],
        0.0,
    )
    o_ref[0] = xp[:, :H]
    o_ref[1] = xp[:, H:]


def _combine_body(agg_lo_ref, agg_hi_ref, cnt0_ref, cnt1_ref, x_ref, wl_ref,
                  bl_ref, wr_ref, o_ref):
    cnt = cnt0_ref[:, 0:1] + cnt1_ref[:, 0:1]
    inv = 1.0 / jnp.maximum(cnt, 1.0)
    m0 = agg_lo_ref[...] * inv
    m1 = agg_hi_ref[...] * inv
    o_ref[...] = (
        jnp.dot(m0, wl_ref[:H, :], preferred_element_type=jnp.float32)
        + jnp.dot(m1, wl_ref[H:, :], preferred_element_type=jnp.float32)
        + jnp.dot(x_ref[...], wr_ref[...], preferred_element_type=jnp.float32)
        + bl_ref[...]
    )


def _zero_agg(sid, z_agg, agg_sh):
    pltpu.sync_copy(z_agg, agg_sh.at[pl.ds(sid * RPT, RPT)])

    @pl.when(sid == 0)
    def _():
        pltpu.sync_copy(z_agg.at[pl.ds(0, TAIL)], agg_sh.at[pl.ds(TAIL0, TAIL)])


def _sc_body(xp_hbm, src_hbm, dst_hbm, z_agg, ones_hbm,
             agg_out, cnt_out,
             src_v, dst_v, rows_v, src_v1, dst_v1, rows_v1, agg_sh,
             sem, sem1):
    cid = lax.axis_index("c")
    sid = lax.axis_index("s")
    coff = jnp.full((L,), cid * N, jnp.int32)

    def load_idx(chunk, src_b, dst_b):
        off = chunk * CH
        pltpu.sync_copy(src_hbm.at[pl.ds(off, CH)], src_b)
        pltpu.sync_copy(dst_hbm.at[pl.ds(off, CH)], dst_b)
        for k in range(CH // L):
            sl = pl.ds(k * L, L)
            src_b[sl] = src_b[sl] + coff

    def gather_scatter_chunk(chunk):
        load_idx(chunk, src_v, dst_v)
        pltpu.async_copy(xp_hbm.at[src_v], rows_v, sem).wait()
        pltpu.sync_copy(rows_v, agg_sh.at[dst_v], add=True)

    def count_chunk(chunk):
        off = chunk * CH
        pltpu.sync_copy(dst_hbm.at[pl.ds(off, CH)], dst_v)
        pltpu.sync_copy(rows_v, agg_sh.at[dst_v], add=True)

    _zero_agg(sid, z_agg, agg_sh)
    plsc.subcore_barrier()

    c1 = sid * CPT1
    HALF = CPT1 // 2
    load_idx(c1, src_v, dst_v)
    pltpu.async_copy(xp_hbm.at[src_v], rows_v, sem)

    @pl.loop(0, HALF)
    def _(jj):
        load_idx(c1 + 2 * jj + 1, src_v1, dst_v1)
        pltpu.async_copy(xp_hbm.at[src_v1], rows_v1, sem1)
        pltpu.make_async_copy(xp_hbm.at[src_v], rows_v, sem).wait()
        pltpu.sync_copy(rows_v, agg_sh.at[dst_v], add=True)

        @pl.when(jj < HALF - 1)
        def _():
            load_idx(c1 + 2 * jj + 2, src_v, dst_v)
            pltpu.async_copy(xp_hbm.at[src_v], rows_v, sem)

        pltpu.make_async_copy(xp_hbm.at[src_v1], rows_v1, sem1).wait()
        pltpu.sync_copy(rows_v1, agg_sh.at[dst_v1], add=True)

    @pl.when(sid < XTRA1)
    def _():
        gather_scatter_chunk(NS * CPT1 + sid)

    plsc.subcore_barrier()

    r0 = sid * RPT
    o0 = cid * N + r0
    pltpu.sync_copy(agg_sh.at[pl.ds(r0, RPT)], agg_out.at[pl.ds(o0, RPT)])

    @pl.when(sid == 0)
    def _():
        pltpu.sync_copy(agg_sh.at[pl.ds(TAIL0, TAIL)],
                        agg_out.at[pl.ds(cid * N + TAIL0, TAIL)])

    plsc.subcore_barrier()

    _zero_agg(sid, z_agg, agg_sh)
    pltpu.sync_copy(ones_hbm, rows_v)
    plsc.subcore_barrier()

    c2 = cid * CPC2 + sid * CPT2

    @pl.loop(0, CPT2)
    def _(j):
        count_chunk(c2 + j)

    @pl.when(sid < XTRA2)
    def _():
        count_chunk(cid * CPC2 + NS * CPT2 + sid)

    plsc.subcore_barrier()

    pltpu.sync_copy(agg_sh.at[pl.ds(r0, RPT)], cnt_out.at[pl.ds(o0, RPT)])

    @pl.when(sid == 0)
    def _():
        pltpu.sync_copy(agg_sh.at[pl.ds(TAIL0, TAIL)],
                        cnt_out.at[pl.ds(cid * N + TAIL0, TAIL)])


_sc_call = pl.kernel(
    _sc_body,
    out_type=[
        jax.ShapeDtypeStruct((NC * N, H), jnp.float32),
        jax.ShapeDtypeStruct((NC * N, H), jnp.float32),
    ],
    mesh=plsc.VectorSubcoreMesh(core_axis_name="c", subcore_axis_name="s"),
    scratch_types=[
        pltpu.VMEM((CH,), jnp.int32),
        pltpu.VMEM((CH,), jnp.int32),
        pltpu.VMEM((CH, H), jnp.float32),
        pltpu.VMEM((CH,), jnp.int32),
        pltpu.VMEM((CH,), jnp.int32),
        pltpu.VMEM((CH, H), jnp.float32),
        pltpu.VMEM_SHARED((N, H), jnp.float32),
        pltpu.SemaphoreType.DMA,
        pltpu.SemaphoreType.DMA,
    ],
)

_ROWS = 1000
_GRID = N // _ROWS

_proj_call = pl.pallas_call(
    _proj_body,
    grid=(_GRID,),
    in_specs=[
        pl.BlockSpec((_ROWS, D), lambda i: (i, 0)),
        pl.BlockSpec((D, D), lambda i: (0, 0)),
        pl.BlockSpec((1, D), lambda i: (0, 0)),
    ],
    out_specs=pl.BlockSpec((NC, _ROWS, H), lambda i: (0, i, 0)),
    out_shape=jax.ShapeDtypeStruct((NC, N, H), jnp.float32),
)

_combine_call = pl.pallas_call(
    _combine_body,
    grid=(_GRID,),
    in_specs=[
        pl.BlockSpec((_ROWS, H), lambda i: (i, 0)),
        pl.BlockSpec((_ROWS, H), lambda i: (i + _GRID, 0)),
        pl.BlockSpec((_ROWS, H), lambda i: (i, 0)),
        pl.BlockSpec((_ROWS, H), lambda i: (i + _GRID, 0)),
        pl.BlockSpec((_ROWS, D), lambda i: (i, 0)),
        pl.BlockSpec((D, D), lambda i: (0, 0)),
        pl.BlockSpec((1, D), lambda i: (0, 0)),
        pl.BlockSpec((D, D), lambda i: (0, 0)),
    ],
    out_specs=pl.BlockSpec((_ROWS, D), lambda i: (i, 0)),
    out_shape=jax.ShapeDtypeStruct((N, D), jnp.float32),
)


@jax.jit
def kernel(x, ei, W_proj, b_proj, W_l, b_l, W_r):
    xp = _proj_call(x, W_proj, b_proj.reshape(1, D))
    xp_flat = xp.reshape(NC * N, H)
    z_agg = jnp.zeros((RPT, H), jnp.float32)
    ones = jnp.ones((CH, H), jnp.float32)
    agg, cnt = _sc_call(xp_flat, ei[0], ei[1], z_agg, ones)
    return _combine_call(agg, agg, cnt, cnt, x, W_l, b_l.reshape(1, D), W_r)


# speedup vs baseline: 6.6342x; 1.1513x over previous
import jax
import jax.numpy as jnp
from jax import lax
from jax.experimental import pallas as pl
from jax.experimental.pallas import tpu as pltpu
from jax.experimental.pallas import tpu_sc as plsc

N = 10000
E = 160000
D = 256
H = 128
NC = 2
NS = 16
CH = 128
NCH = E // CH
CPT1 = NCH // NS
XTRA1 = NCH - CPT1 * NS
CPC2 = NCH // NC
CPT2 = CPC2 // NS
XTRA2 = CPC2 - CPT2 * NS
RPT = 624
TAIL = N - NS * RPT
TAIL0 = N - TAIL

B1 = 6
NB1 = CPT1 // B1
IB1 = B1 * CH
B2 = 13
NB2 = CPT2 // B2
IB2 = B2 * CH


def _proj_body(x_ref, wp_ref, bp_ref, o_ref):
    xp = jnp.maximum(
        jnp.dot(x_ref[...], wp_ref[...], preferred_element_type=jnp.float32)
        + bp_ref[...],
        0.0,
    )
    o_ref[0] = xp[:, :H]
    o_ref[1] = xp[:, H:]


def _combine_body(agg_lo_ref, agg_hi_ref, cnt0_ref, cnt1_ref, x_ref, wl_ref,
                  bl_ref, wr_ref, o_ref):
    cnt = cnt0_ref[:, 0:1] + cnt1_ref[:, 0:1]
    inv = 1.0 / jnp.maximum(cnt, 1.0)
    m0 = agg_lo_ref[...] * inv
    m1 = agg_hi_ref[...] * inv
    o_ref[...] = (
        jnp.dot(m0, wl_ref[:H, :], preferred_element_type=jnp.float32)
        + jnp.dot(m1, wl_ref[H:, :], preferred_element_type=jnp.float32)
        + jnp.dot(x_ref[...], wr_ref[...], preferred_element_type=jnp.float32)
        + bl_ref[...]
    )


def _zero_agg(sid, z_agg, agg_sh):
    pltpu.sync_copy(z_agg, agg_sh.at[pl.ds(sid * RPT, RPT)])

    @pl.when(sid == 0)
    def _():
        pltpu.sync_copy(z_agg.at[pl.ds(0, TAIL)], agg_sh.at[pl.ds(TAIL0, TAIL)])


def _sc_body(xp_hbm, eix_hbm, z_agg, ones_hbm,
             agg_out, cnt_out,
             idxA, idxB, cbuf, rows_v, rows_v1, agg_sh,
             sem, sem1):
    cid = lax.axis_index("c")
    sid = lax.axis_index("s")
    rows = (rows_v, rows_v1)
    sems = (sem, sem1)

    _zero_agg(sid, z_agg, agg_sh)
    plsc.subcore_barrier()

    base1 = cid * E + sid * (CPT1 * CH)

    def gather(buf, j, r):
        pltpu.async_copy(xp_hbm.at[buf.at[0, pl.ds(j * CH, CH)]],
                         rows[r], sems[r])

    def finish_chunk(buf, j, r):
        pltpu.make_async_copy(xp_hbm.at[buf.at[0, pl.ds(j * CH, CH)]],
                              rows[r], sems[r]).wait()
        pltpu.sync_copy(rows[r], agg_sh.at[buf.at[1, pl.ds(j * CH, CH)]],
                        add=True)

    def run_block(buf, nextbuf):
        for j in range(B1):
            if j + 1 < B1:
                gather(buf, j + 1, (j + 1) % 2)
            else:
                gather(nextbuf, 0, 0)
            finish_chunk(buf, j, j % 2)

    pltpu.sync_copy(eix_hbm.at[:, pl.ds(base1, IB1)], idxA)
    gather(idxA, 0, 0)

    @pl.loop(0, NB1 // 2)
    def _(bp):
        colB = base1 + (2 * bp + 1) * IB1
        pltpu.sync_copy(eix_hbm.at[:, pl.ds(colB, IB1)], idxB)
        run_block(idxA, idxB)
        colA = base1 + (2 * bp + 2) * IB1
        pltpu.sync_copy(eix_hbm.at[:, pl.ds(colA, IB1)], idxA)
        run_block(idxB, idxA)

    for j in range(B1):
        if j + 1 < B1:
            gather(idxA, j + 1, (j + 1) % 2)
        finish_chunk(idxA, j, j % 2)

    @pl.when(sid < XTRA1)
    def _():
        col = cid * E + (NS * CPT1 + sid) * CH
        pltpu.sync_copy(eix_hbm.at[:, pl.ds(col, CH)], idxA.at[:, pl.ds(0, CH)])
        gather(idxA, 0, 0)
        finish_chunk(idxA, 0, 0)

    plsc.subcore_barrier()

    r0 = sid * RPT
    o0 = cid * N + r0
    pltpu.sync_copy(agg_sh.at[pl.ds(r0, RPT)], agg_out.at[pl.ds(o0, RPT)])

    @pl.when(sid == 0)
    def _():
        pltpu.sync_copy(agg_sh.at[pl.ds(TAIL0, TAIL)],
                        agg_out.at[pl.ds(cid * N + TAIL0, TAIL)])

    plsc.subcore_barrier()

    _zero_agg(sid, z_agg, agg_sh)
    pltpu.sync_copy(ones_hbm, rows_v)
    plsc.subcore_barrier()

    c2base = (cid * CPC2 + sid * CPT2) * CH

    @pl.loop(0, NB2)
    def _(b):
        pltpu.sync_copy(eix_hbm.at[1, pl.ds(c2base + b * IB2, IB2)], cbuf)

        @pl.loop(0, B2)
        def _(j):
            pltpu.sync_copy(rows_v, agg_sh.at[cbuf.at[pl.ds(j * CH, CH)]],
                            add=True)

    @pl.when(sid < XTRA2)
    def _():
        col = (cid * CPC2 + NS * CPT2 + sid) * CH
        pltpu.sync_copy(eix_hbm.at[1, pl.ds(col, CH)], cbuf.at[pl.ds(0, CH)])
        pltpu.sync_copy(rows_v, agg_sh.at[cbuf.at[pl.ds(0, CH)]], add=True)

    plsc.subcore_barrier()

    pltpu.sync_copy(agg_sh.at[pl.ds(r0, RPT)], cnt_out.at[pl.ds(o0, RPT)])

    @pl.when(sid == 0)
    def _():
        pltpu.sync_copy(agg_sh.at[pl.ds(TAIL0, TAIL)],
                        cnt_out.at[pl.ds(cid * N + TAIL0, TAIL)])


_sc_call = pl.kernel(
    _sc_body,
    out_type=[
        jax.ShapeDtypeStruct((NC * N, H), jnp.float32),
        jax.ShapeDtypeStruct((NC * N, H), jnp.float32),
    ],
    mesh=plsc.VectorSubcoreMesh(core_axis_name="c", subcore_axis_name="s"),
    scratch_types=[
        pltpu.VMEM((2, IB1), jnp.int32),
        pltpu.VMEM((2, IB1), jnp.int32),
        pltpu.VMEM((IB2,), jnp.int32),
        pltpu.VMEM((CH, H), jnp.float32),
        pltpu.VMEM((CH, H), jnp.float32),
        pltpu.VMEM_SHARED((N, H), jnp.float32),
        pltpu.SemaphoreType.DMA,
        pltpu.SemaphoreType.DMA,
    ],
)

_ROWS = 1000
_GRID = N // _ROWS

_proj_call = pl.pallas_call(
    _proj_body,
    grid=(_GRID,),
    in_specs=[
        pl.BlockSpec((_ROWS, D), lambda i: (i, 0)),
        pl.BlockSpec((D, D), lambda i: (0, 0)),
        pl.BlockSpec((1, D), lambda i: (0, 0)),
    ],
    out_specs=pl.BlockSpec((NC, _ROWS, H), lambda i: (0, i, 0)),
    out_shape=jax.ShapeDtypeStruct((NC, N, H), jnp.float32),
)

_combine_call = pl.pallas_call(
    _combine_body,
    grid=(_GRID,),
    in_specs=[
        pl.BlockSpec((_ROWS, H), lambda i: (i, 0)),
        pl.BlockSpec((_ROWS, H), lambda i: (i + _GRID, 0)),
        pl.BlockSpec((_ROWS, H), lambda i: (i, 0)),
        pl.BlockSpec((_ROWS, H), lambda i: (i + _GRID, 0)),
        pl.BlockSpec((_ROWS, D), lambda i: (i, 0)),
        pl.BlockSpec((D, D), lambda i: (0, 0)),
        pl.BlockSpec((1, D), lambda i: (0, 0)),
        pl.BlockSpec((D, D), lambda i: (0, 0)),
    ],
    out_specs=pl.BlockSpec((_ROWS, D), lambda i: (i, 0)),
    out_shape=jax.ShapeDtypeStruct((N, D), jnp.float32),
)


@jax.jit
def kernel(x, ei, W_proj, b_proj, W_l, b_l, W_r):
    xp = _proj_call(x, W_proj, b_proj.reshape(1, D))
    xp_flat = xp.reshape(NC * N, H)
    src = ei[0].astype(jnp.int32)
    dst = ei[1].astype(jnp.int32)
    eix = jnp.stack([
        jnp.concatenate([src, src + N]),
        jnp.concatenate([dst, dst]),
    ])
    z_agg = jnp.zeros((RPT, H), jnp.float32)
    ones = jnp.ones((CH, H), jnp.float32)
    agg, cnt = _sc_call(xp_flat, eix, z_agg, ones)
    return _combine_call(agg, agg, cnt, cnt, x, W_l, b_l.reshape(1, D), W_r)


# speedup vs baseline: 6.7497x; 1.0174x over previous
import jax
import jax.numpy as jnp
from jax import lax
from jax.experimental import pallas as pl
from jax.experimental.pallas import tpu as pltpu
from jax.experimental.pallas import tpu_sc as plsc

N = 10000
E = 160000
D = 256
H = 128
NC = 2
NS = 16
CH = 128
NCH = E // CH
CPT1 = NCH // NS
XTRA1 = NCH - CPT1 * NS
CPC2 = NCH // NC
CPT2 = CPC2 // NS
XTRA2 = CPC2 - CPT2 * NS
RPT = 624
TAIL = N - NS * RPT
TAIL0 = N - TAIL

B1 = 26
NB1 = CPT1 // B1
IB1 = B1 * CH
P2A = B1
P2B = CPT2 - P2A


def _proj_body(x_ref, wp_ref, bp_ref, o_ref):
    xp = jnp.maximum(
        jnp.dot(x_ref[...], wp_ref[...], preferred_element_type=jnp.float32)
        + bp_ref[...],
        0.0,
    )
    o_ref[0] = xp[:, :H]
    o_ref[1] = xp[:, H:]


def _xr_body(x_ref, wr_ref, bl_ref, o_ref):
    o_ref[...] = (
        jnp.dot(x_ref[...], wr_ref[...], preferred_element_type=jnp.float32)
        + bl_ref[...]
    )


def _combine_body(agg_lo_ref, agg_hi_ref, cnt0_ref, cnt1_ref, xr_ref, wl_ref,
                  o_ref):
    cnt = cnt0_ref[:, 0:1] + cnt1_ref[:, 0:1]
    inv = 1.0 / jnp.maximum(cnt, 1.0)
    m0 = agg_lo_ref[...] * inv
    m1 = agg_hi_ref[...] * inv
    o_ref[...] = (
        jnp.dot(m0, wl_ref[:H, :], preferred_element_type=jnp.float32)
        + jnp.dot(m1, wl_ref[H:, :], preferred_element_type=jnp.float32)
        + xr_ref[...]
    )


def _zero_agg(sid, z_agg, agg_sh):
    pltpu.sync_copy(z_agg, agg_sh.at[pl.ds(sid * RPT, RPT)])

    @pl.when(sid == 0)
    def _():
        pltpu.sync_copy(z_agg.at[pl.ds(0, TAIL)], agg_sh.at[pl.ds(TAIL0, TAIL)])


def _sc_body(xp_hbm, eix_hbm, z_agg, ones_hbm,
             agg_out, cnt_out,
             idxA, idxB, rows_v, rows_v1, agg_sh,
             sem, sem1, ssem):
    cid = lax.axis_index("c")
    sid = lax.axis_index("s")
    rows = (rows_v, rows_v1)
    sems = (sem, sem1)

    _zero_agg(sid, z_agg, agg_sh)
    plsc.subcore_barrier()

    base1 = cid * E + sid * (CPT1 * CH)

    def gather(buf, j, r):
        pltpu.async_copy(xp_hbm.at[buf.at[0, pl.ds(j * CH, CH)]],
                         rows[r], sems[r])

    def finish_chunk(buf, j, r):
        pltpu.make_async_copy(xp_hbm.at[buf.at[0, pl.ds(j * CH, CH)]],
                              rows[r], sems[r]).wait()
        pltpu.sync_copy(rows[r], agg_sh.at[buf.at[1, pl.ds(j * CH, CH)]],
                        add=True)

    def run_block(buf, nextbuf):
        for j in range(B1):
            if j + 1 < B1:
                gather(buf, j + 1, (j + 1) % 2)
            else:
                gather(nextbuf, 0, 0)
            finish_chunk(buf, j, j % 2)

    pltpu.sync_copy(eix_hbm.at[:, pl.ds(base1, IB1)], idxA)
    gather(idxA, 0, 0)
    pltpu.sync_copy(eix_hbm.at[:, pl.ds(base1 + IB1, IB1)], idxB)
    run_block(idxA, idxB)
    pltpu.sync_copy(eix_hbm.at[:, pl.ds(base1 + 2 * IB1, IB1)], idxA)
    run_block(idxB, idxA)
    for j in range(B1):
        if j + 1 < B1:
            gather(idxA, j + 1, (j + 1) % 2)
        finish_chunk(idxA, j, j % 2)

    @pl.when(sid < XTRA1)
    def _():
        col = cid * E + (NS * CPT1 + sid) * CH
        pltpu.sync_copy(eix_hbm.at[:, pl.ds(col, CH)], idxA.at[:, pl.ds(0, CH)])
        gather(idxA, 0, 0)
        finish_chunk(idxA, 0, 0)

    plsc.subcore_barrier()

    r0 = sid * RPT
    o0 = cid * N + r0
    pltpu.sync_copy(agg_sh.at[pl.ds(r0, RPT)], agg_out.at[pl.ds(o0, RPT)])

    @pl.when(sid == 0)
    def _():
        pltpu.sync_copy(agg_sh.at[pl.ds(TAIL0, TAIL)],
                        agg_out.at[pl.ds(cid * N + TAIL0, TAIL)])

    plsc.subcore_barrier()

    _zero_agg(sid, z_agg, agg_sh)
    pltpu.sync_copy(ones_hbm, rows_v)
    plsc.subcore_barrier()

    c2base = (cid * CPC2 + sid * CPT2) * CH
    pltpu.sync_copy(eix_hbm.at[1, pl.ds(c2base, P2A * CH)], idxA.at[0])
    pltpu.sync_copy(eix_hbm.at[1, pl.ds(c2base + P2A * CH, P2B * CH)],
                    idxA.at[1, pl.ds(0, P2B * CH)])

    @pl.loop(0, P2A)
    def _(j):
        pltpu.async_copy(rows_v, agg_sh.at[idxA.at[0, pl.ds(j * CH, CH)]],
                         ssem, add=True)

    @pl.loop(0, P2B)
    def _(j):
        pltpu.async_copy(rows_v, agg_sh.at[idxA.at[1, pl.ds(j * CH, CH)]],
                         ssem, add=True)

    @pl.when(sid < XTRA2)
    def _():
        col = (cid * CPC2 + NS * CPT2 + sid) * CH
        pltpu.sync_copy(eix_hbm.at[1, pl.ds(col, CH)],
                        idxB.at[0, pl.ds(0, CH)])
        pltpu.async_copy(rows_v, agg_sh.at[idxB.at[0, pl.ds(0, CH)]],
                         ssem, add=True)

    @pl.loop(0, P2A)
    def _(j):
        pltpu.make_async_copy(rows_v, agg_sh.at[idxA.at[0, pl.ds(j * CH, CH)]],
                              ssem).wait()

    @pl.loop(0, P2B)
    def _(j):
        pltpu.make_async_copy(rows_v, agg_sh.at[idxA.at[1, pl.ds(j * CH, CH)]],
                              ssem).wait()

    @pl.when(sid < XTRA2)
    def _():
        pltpu.make_async_copy(rows_v, agg_sh.at[idxB.at[0, pl.ds(0, CH)]],
                              ssem).wait()

    plsc.subcore_barrier()

    pltpu.sync_copy(agg_sh.at[pl.ds(r0, RPT)], cnt_out.at[pl.ds(o0, RPT)])

    @pl.when(sid == 0)
    def _():
        pltpu.sync_copy(agg_sh.at[pl.ds(TAIL0, TAIL)],
                        cnt_out.at[pl.ds(cid * N + TAIL0, TAIL)])


_sc_call = pl.kernel(
    _sc_body,
    out_type=[
        jax.ShapeDtypeStruct((NC * N, H), jnp.float32),
        jax.ShapeDtypeStruct((NC * N, H), jnp.float32),
    ],
    mesh=plsc.VectorSubcoreMesh(core_axis_name="c", subcore_axis_name="s"),
    scratch_types=[
        pltpu.VMEM((2, IB1), jnp.int32),
        pltpu.VMEM((2, IB1), jnp.int32),
        pltpu.VMEM((CH, H), jnp.float32),
        pltpu.VMEM((CH, H), jnp.float32),
        pltpu.VMEM_SHARED((N, H), jnp.float32),
        pltpu.SemaphoreType.DMA,
        pltpu.SemaphoreType.DMA,
        pltpu.SemaphoreType.DMA,
    ],
)

_ROWS = 1000
_GRID = N // _ROWS

_proj_call = pl.pallas_call(
    _proj_body,
    grid=(_GRID,),
    in_specs=[
        pl.BlockSpec((_ROWS, D), lambda i: (i, 0)),
        pl.BlockSpec((D, D), lambda i: (0, 0)),
        pl.BlockSpec((1, D), lambda i: (0, 0)),
    ],
    out_specs=pl.BlockSpec((NC, _ROWS, H), lambda i: (0, i, 0)),
    out_shape=jax.ShapeDtypeStruct((NC, N, H), jnp.float32),
)

_xr_call = pl.pallas_call(
    _xr_body,
    grid=(_GRID,),
    in_specs=[
        pl.BlockSpec((_ROWS, D), lambda i: (i, 0)),
        pl.BlockSpec((D, D), lambda i: (0, 0)),
        pl.BlockSpec((1, D), lambda i: (0, 0)),
    ],
    out_specs=pl.BlockSpec((_ROWS, D), lambda i: (i, 0)),
    out_shape=jax.ShapeDtypeStruct((N, D), jnp.float32),
)

_combine_call = pl.pallas_call(
    _combine_body,
    grid=(_GRID,),
    in_specs=[
        pl.BlockSpec((_ROWS, H), lambda i: (i, 0)),
        pl.BlockSpec((_ROWS, H), lambda i: (i + _GRID, 0)),
        pl.BlockSpec((_ROWS, H), lambda i: (i, 0)),
        pl.BlockSpec((_ROWS, H), lambda i: (i + _GRID, 0)),
        pl.BlockSpec((_ROWS, D), lambda i: (i, 0)),
        pl.BlockSpec((D, D), lambda i: (0, 0)),
    ],
    out_specs=pl.BlockSpec((_ROWS, D), lambda i: (i, 0)),
    out_shape=jax.ShapeDtypeStruct((N, D), jnp.float32),
)


@jax.jit
def kernel(x, ei, W_proj, b_proj, W_l, b_l, W_r):
    xp = _proj_call(x, W_proj, b_proj.reshape(1, D))
    xr = _xr_call(x, W_r, b_l.reshape(1, D))
    xp_flat = xp.reshape(NC * N, H)
    src = ei[0].astype(jnp.int32)
    dst = ei[1].astype(jnp.int32)
    eix = jnp.stack([
        jnp.concatenate([src, src + N]),
        jnp.concatenate([dst, dst]),
    ])
    z_agg = jnp.zeros((RPT, H), jnp.float32)
    ones = jnp.ones((CH, H), jnp.float32)
    agg, cnt = _sc_call(xp_flat, eix, z_agg, ones)
    return _combine_call(agg, agg, cnt, cnt, xr, W_l)


# speedup vs baseline: 6.7981x; 1.0072x over previous
import jax
import jax.numpy as jnp
from jax import lax
from jax.experimental import pallas as pl
from jax.experimental.pallas import tpu as pltpu
from jax.experimental.pallas import tpu_sc as plsc

N = 10000
E = 160000
D = 256
H = 128
NC = 2
NS = 16
CH = 128
NCH = E // CH
CPT1 = NCH // NS
XTRA1 = NCH - CPT1 * NS
CPC2 = NCH // NC
CPT2 = CPC2 // NS
XTRA2 = CPC2 - CPT2 * NS
RPT = 624
TAIL = N - NS * RPT
TAIL0 = N - TAIL

B1 = 26
NB1 = CPT1 // B1
IB1 = B1 * CH
P2A = B1
P2B = CPT2 - P2A


def _pre_body(x_ref, wp_ref, bp_ref, wr_ref, bl_ref, xp_ref, xr_ref):
    xb = x_ref[...]
    xp = jnp.maximum(
        jnp.dot(xb, wp_ref[...], preferred_element_type=jnp.float32)
        + bp_ref[...],
        0.0,
    )
    xp_ref[0] = xp[:, :H]
    xp_ref[1] = xp[:, H:]
    xr_ref[...] = (
        jnp.dot(xb, wr_ref[...], preferred_element_type=jnp.float32)
        + bl_ref[...]
    )


def _combine_body(agg_lo_ref, agg_hi_ref, cnt0_ref, cnt1_ref, xr_ref, wl_ref,
                  o_ref):
    cnt = cnt0_ref[:, 0:1] + cnt1_ref[:, 0:1]
    inv = 1.0 / jnp.maximum(cnt, 1.0)
    m0 = agg_lo_ref[...] * inv
    m1 = agg_hi_ref[...] * inv
    o_ref[...] = (
        jnp.dot(m0, wl_ref[:H, :], preferred_element_type=jnp.float32)
        + jnp.dot(m1, wl_ref[H:, :], preferred_element_type=jnp.float32)
        + xr_ref[...]
    )


def _zero_agg(sid, z_agg, agg_sh):
    pltpu.sync_copy(z_agg, agg_sh.at[pl.ds(sid * RPT, RPT)])

    @pl.when(sid == 0)
    def _():
        pltpu.sync_copy(z_agg.at[pl.ds(0, TAIL)], agg_sh.at[pl.ds(TAIL0, TAIL)])


def _sc_body(xp_hbm, eix_hbm, z_agg, ones_hbm,
             agg_out, cnt_out,
             idxA, idxB, rows_v, rows_v1, agg_sh,
             g0, g1, s0, s1):
    cid = lax.axis_index("c")
    sid = lax.axis_index("s")
    rows = (rows_v, rows_v1)
    gsems = (g0, g1)
    ssems = (s0, s1)

    _zero_agg(sid, z_agg, agg_sh)
    plsc.subcore_barrier()

    base1 = cid * E + sid * (CPT1 * CH)

    def gather(buf, j, r):
        pltpu.async_copy(xp_hbm.at[buf.at[0, pl.ds(j * CH, CH)]],
                         rows[r], gsems[r])

    def wait_gather(buf, j, r):
        pltpu.make_async_copy(xp_hbm.at[buf.at[0, pl.ds(j * CH, CH)]],
                              rows[r], gsems[r]).wait()

    def scatter(buf, j, r):
        pltpu.async_copy(rows[r], agg_sh.at[buf.at[1, pl.ds(j * CH, CH)]],
                         ssems[r], add=True)

    def wait_scatter(buf, j, r):
        pltpu.make_async_copy(rows[r], agg_sh.at[buf.at[1, pl.ds(j * CH, CH)]],
                              ssems[r]).wait()

    seq = ([(idxA, j) for j in range(B1)]
           + [(idxB, j) for j in range(B1)]
           + [(idxA, j) for j in range(B1)])
    pending = [None, None]

    pltpu.sync_copy(eix_hbm.at[:, pl.ds(base1, IB1)], idxA)
    gather(*seq[0], 0)
    pltpu.sync_copy(eix_hbm.at[:, pl.ds(base1 + IB1, IB1)], idxB)

    for t in range(CPT1):
        r = t % 2
        if t + 1 < CPT1:
            rn = (t + 1) % 2
            if pending[rn] is not None:
                wait_scatter(*pending[rn], rn)
                pending[rn] = None
            gather(*seq[t + 1], rn)
        wait_gather(*seq[t], r)
        scatter(*seq[t], r)
        pending[r] = seq[t]
        if t == B1:
            pltpu.sync_copy(eix_hbm.at[:, pl.ds(base1 + 2 * IB1, IB1)], idxA)

    for r in range(2):
        if pending[r] is not None:
            wait_scatter(*pending[r], r)

    @pl.when(sid < XTRA1)
    def _():
        col = cid * E + (NS * CPT1 + sid) * CH
        pltpu.sync_copy(eix_hbm.at[:, pl.ds(col, CH)], idxA.at[:, pl.ds(0, CH)])
        gather(idxA, 0, 0)
        wait_gather(idxA, 0, 0)
        pltpu.sync_copy(rows_v, agg_sh.at[idxA.at[1, pl.ds(0, CH)]], add=True)

    plsc.subcore_barrier()

    r0 = sid * RPT
    o0 = cid * N + r0
    pltpu.sync_copy(agg_sh.at[pl.ds(r0, RPT)], agg_out.at[pl.ds(o0, RPT)])

    @pl.when(sid == 0)
    def _():
        pltpu.sync_copy(agg_sh.at[pl.ds(TAIL0, TAIL)],
                        agg_out.at[pl.ds(cid * N + TAIL0, TAIL)])

    plsc.subcore_barrier()

    _zero_agg(sid, z_agg, agg_sh)
    pltpu.sync_copy(ones_hbm, rows_v)
    plsc.subcore_barrier()

    c2base = (cid * CPC2 + sid * CPT2) * CH
    pltpu.sync_copy(eix_hbm.at[1, pl.ds(c2base, P2A * CH)], idxA.at[0])
    pltpu.sync_copy(eix_hbm.at[1, pl.ds(c2base + P2A * CH, P2B * CH)],
                    idxA.at[1, pl.ds(0, P2B * CH)])

    @pl.loop(0, P2A)
    def _(j):
        pltpu.async_copy(rows_v, agg_sh.at[idxA.at[0, pl.ds(j * CH, CH)]],
                         s0, add=True)

    @pl.loop(0, P2B)
    def _(j):
        pltpu.async_copy(rows_v, agg_sh.at[idxA.at[1, pl.ds(j * CH, CH)]],
                         s0, add=True)

    @pl.when(sid < XTRA2)
    def _():
        col = (cid * CPC2 + NS * CPT2 + sid) * CH
        pltpu.sync_copy(eix_hbm.at[1, pl.ds(col, CH)],
                        idxB.at[0, pl.ds(0, CH)])
        pltpu.async_copy(rows_v, agg_sh.at[idxB.at[0, pl.ds(0, CH)]],
                         s0, add=True)

    @pl.loop(0, P2A)
    def _(j):
        pltpu.make_async_copy(rows_v, agg_sh.at[idxA.at[0, pl.ds(j * CH, CH)]],
                              s0).wait()

    @pl.loop(0, P2B)
    def _(j):
        pltpu.make_async_copy(rows_v, agg_sh.at[idxA.at[1, pl.ds(j * CH, CH)]],
                              s0).wait()

    @pl.when(sid < XTRA2)
    def _():
        pltpu.make_async_copy(rows_v, agg_sh.at[idxB.at[0, pl.ds(0, CH)]],
                              s0).wait()

    plsc.subcore_barrier()

    pltpu.sync_copy(agg_sh.at[pl.ds(r0, RPT)], cnt_out.at[pl.ds(o0, RPT)])

    @pl.when(sid == 0)
    def _():
        pltpu.sync_copy(agg_sh.at[pl.ds(TAIL0, TAIL)],
                        cnt_out.at[pl.ds(cid * N + TAIL0, TAIL)])


_sc_call = pl.kernel(
    _sc_body,
    out_type=[
        jax.ShapeDtypeStruct((NC * N, H), jnp.float32),
        jax.ShapeDtypeStruct((NC * N, H), jnp.float32),
    ],
    mesh=plsc.VectorSubcoreMesh(core_axis_name="c", subcore_axis_name="s"),
    scratch_types=[
        pltpu.VMEM((2, IB1), jnp.int32),
        pltpu.VMEM((2, IB1), jnp.int32),
        pltpu.VMEM((CH, H), jnp.float32),
        pltpu.VMEM((CH, H), jnp.float32),
        pltpu.VMEM_SHARED((N, H), jnp.float32),
        pltpu.SemaphoreType.DMA,
        pltpu.SemaphoreType.DMA,
        pltpu.SemaphoreType.DMA,
        pltpu.SemaphoreType.DMA,
    ],
)

_ROWS = 1000
_GRID = N // _ROWS

_pre_call = pl.pallas_call(
    _pre_body,
    grid=(_GRID,),
    in_specs=[
        pl.BlockSpec((_ROWS, D), lambda i: (i, 0)),
        pl.BlockSpec((D, D), lambda i: (0, 0)),
        pl.BlockSpec((1, D), lambda i: (0, 0)),
        pl.BlockSpec((D, D), lambda i: (0, 0)),
        pl.BlockSpec((1, D), lambda i: (0, 0)),
    ],
    out_specs=[
        pl.BlockSpec((NC, _ROWS, H), lambda i: (0, i, 0)),
        pl.BlockSpec((_ROWS, D), lambda i: (i, 0)),
    ],
    out_shape=[
        jax.ShapeDtypeStruct((NC, N, H), jnp.float32),
        jax.ShapeDtypeStruct((N, D), jnp.float32),
    ],
)

_combine_call = pl.pallas_call(
    _combine_body,
    grid=(_GRID,),
    in_specs=[
        pl.BlockSpec((_ROWS, H), lambda i: (i, 0)),
        pl.BlockSpec((_ROWS, H), lambda i: (i + _GRID, 0)),
        pl.BlockSpec((_ROWS, H), lambda i: (i, 0)),
        pl.BlockSpec((_ROWS, H), lambda i: (i + _GRID, 0)),
        pl.BlockSpec((_ROWS, D), lambda i: (i, 0)),
        pl.BlockSpec((D, D), lambda i: (0, 0)),
    ],
    out_specs=pl.BlockSpec((_ROWS, D), lambda i: (i, 0)),
    out_shape=jax.ShapeDtypeStruct((N, D), jnp.float32),
)


@jax.jit
def kernel(x, ei, W_proj, b_proj, W_l, b_l, W_r):
    xp, xr = _pre_call(x, W_proj, b_proj.reshape(1, D), W_r, b_l.reshape(1, D))
    xp_flat = xp.reshape(NC * N, H)
    src = ei[0].astype(jnp.int32)
    dst = ei[1].astype(jnp.int32)
    eix = jnp.stack([
        jnp.concatenate([src, src + N]),
        jnp.concatenate([dst, dst]),
    ])
    z_agg = jnp.zeros((RPT, H), jnp.float32)
    ones = jnp.ones((CH, H), jnp.float32)
    agg, cnt = _sc_call(xp_flat, eix, z_agg, ones)
    return _combine_call(agg, agg, cnt, cnt, xr, W_l)
